# Initial kernel scaffold; baseline (speedup 1.0000x reference)
import functools

import jax
import jax.numpy as jnp
from jax import lax
from jax.experimental import pallas as pl
from jax.experimental.pallas import tpu as pltpu
from jax.experimental.pallas import tpu_sc as plsc

_N = 10000
_E = 320000
_D = 128

_NC = 2
_NS = 16
_NW = _NC * _NS

_CH = 128
_NCHUNK = 80
_EPT = _CH * _NCHUNK
_EPAD = _EPT * _NW
_RPT = _N // _NS
_ZROWS = 125

_mesh = plsc.VectorSubcoreMesh(core_axis_name="c", subcore_axis_name="s")


@functools.partial(
    pl.kernel,
    out_type=jax.ShapeDtypeStruct((_NC, _N, 16), jnp.float32),
    mesh=_mesh,
    scratch_types=[
        pltpu.VMEM((_NCHUNK, _CH), jnp.int32),
        pltpu.VMEM((_NCHUNK, _CH), jnp.float32),
        pltpu.VMEM((_CH, 16), jnp.float32),
        pltpu.VMEM_SHARED((_N, 16), jnp.float32),
        pltpu.SemaphoreType.DMA,
    ],
)
def _deg_sc(dst_hbm, ew_hbm, out_hbm, dstv, ewv, rows, acc, sem):
    cid = lax.axis_index("c")
    sid = lax.axis_index("s")
    wid = cid * _NS + sid

    @pl.loop(0, _ZROWS)
    def _(i):
        rows[i, :] = jnp.zeros((16,), jnp.float32)

    @pl.loop(0, _RPT // _ZROWS)
    def _(j):
        pltpu.sync_copy(
            rows.at[pl.ds(0, _ZROWS)],
            acc.at[pl.ds(sid * _RPT + j * _ZROWS, _ZROWS)],
        )

    plsc.subcore_barrier()

    pltpu.sync_copy(dst_hbm.at[wid], dstv)
    pltpu.sync_copy(ew_hbm.at[wid], ewv)

    @pl.loop(0, _NCHUNK)
    def _(k):
        @pl.loop(0, _CH)
        def _(e):
            w = plsc.load_gather(
                ewv,
                [jnp.full((16,), k, jnp.int32), jnp.full((16,), e, jnp.int32)],
            )
            rows[e, :] = (w + 1.0) * 0.5

        pltpu.sync_copy(rows, acc.at[dstv.at[k]], add=True)

    plsc.subcore_barrier()
    pltpu.sync_copy(
        acc.at[pl.ds(sid * _RPT, _RPT)],
        out_hbm.at[cid, pl.ds(sid * _RPT, _RPT)],
    )


@functools.partial(
    pl.kernel,
    out_type=jax.ShapeDtypeStruct((_NC, _N, _D), jnp.float32),
    mesh=_mesh,
    scratch_types=[
        pltpu.VMEM((_NCHUNK, _CH), jnp.int32),
        pltpu.VMEM((_NCHUNK, _CH), jnp.int32),
        pltpu.VMEM((_NCHUNK, _CH), jnp.float32),
        pltpu.VMEM((_CH, _D), jnp.float32),
        pltpu.VMEM_SHARED((_N, _D), jnp.float32),
        pltpu.SemaphoreType.DMA,
    ],
)
def _agg_sc(src_hbm, dst_hbm, ew_hbm, y_hbm, out_hbm,
            srcv, dstv, ewv, rows, acc, sem):
    cid = lax.axis_index("c")
    sid = lax.axis_index("s")
    wid = cid * _NS + sid

    @pl.loop(0, _ZROWS)
    def _(i):
        for j in range(_D // 16):
            rows[i, pl.ds(j * 16, 16)] = jnp.zeros((16,), jnp.float32)

    @pl.loop(0, _RPT // _ZROWS)
    def _(j):
        pltpu.sync_copy(
            rows.at[pl.ds(0, _ZROWS)],
            acc.at[pl.ds(sid * _RPT + j * _ZROWS, _ZROWS)],
        )

    plsc.subcore_barrier()

    pltpu.sync_copy(src_hbm.at[wid], srcv)
    pltpu.sync_copy(dst_hbm.at[wid], dstv)
    pltpu.sync_copy(ew_hbm.at[wid], ewv)

    @pl.loop(0, _NCHUNK)
    def _(k):
        pltpu.async_copy(y_hbm.at[srcv.at[k]], rows, sem).wait()

        @pl.loop(0, _CH)
        def _(e):
            w = plsc.load_gather(
                ewv,
                [jnp.full((16,), k, jnp.int32), jnp.full((16,), e, jnp.int32)],
            )
            w = (w + 1.0) * 0.5
            for j in range(_D // 16):
                sl = pl.ds(j * 16, 16)
                rows[e, sl] = rows[e, sl] * w

        pltpu.sync_copy(rows, acc.at[dstv.at[k]], add=True)

    plsc.subcore_barrier()
    pltpu.sync_copy(
        acc.at[pl.ds(sid * _RPT, _RPT)],
        out_hbm.at[cid, pl.ds(sid * _RPT, _RPT)],
    )


_BLK = 1000


def _y_body(degp_ref, x_ref, y_ref):
    deg = 1.0 + degp_ref[0, :, 0:1] + degp_ref[1, :, 0:1]
    y_ref[...] = x_ref[...] * lax.rsqrt(deg)


def _y_tc(degp, x):
    return pl.pallas_call(
        _y_body,
        grid=(_N // _BLK,),
        in_specs=[
            pl.BlockSpec((_NC, _BLK, 16), lambda i: (0, i, 0)),
            pl.BlockSpec((_BLK, _D), lambda i: (i, 0)),
        ],
        out_specs=pl.BlockSpec((_BLK, _D), lambda i: (i, 0)),
        out_shape=jax.ShapeDtypeStruct((_N, _D), jnp.float32),
    )(degp, x)


def _out_body(accp_ref, degp_ref, x_ref, w_ref, b_ref, g_ref, bt_ref, o_ref):
    deg = 1.0 + degp_ref[0, :, 0:1] + degp_ref[1, :, 0:1]
    dinv = lax.rsqrt(deg)
    z = (accp_ref[0] + accp_ref[1]) * dinv + x_ref[...] * (1.0 / deg)
    h = jnp.dot(z, w_ref[...], preferred_element_type=jnp.float32) + b_ref[...]
    h = jnp.where(h >= 0.0, h, 0.01 * h)
    mu = jnp.mean(h, axis=-1, keepdims=True)
    var = jnp.mean((h - mu) ** 2, axis=-1, keepdims=True)
    o_ref[...] = (h - mu) * lax.rsqrt(var + 1e-5) * g_ref[...] + bt_ref[...]


def _out_tc(accp, degp, x, W, b, gamma, beta):
    return pl.pallas_call(
        _out_body,
        grid=(_N // _BLK,),
        in_specs=[
            pl.BlockSpec((_NC, _BLK, _D), lambda i: (0, i, 0)),
            pl.BlockSpec((_NC, _BLK, 16), lambda i: (0, i, 0)),
            pl.BlockSpec((_BLK, _D), lambda i: (i, 0)),
            pl.BlockSpec((_D, _D), lambda i: (0, 0)),
            pl.BlockSpec((1, _D), lambda i: (0, 0)),
            pl.BlockSpec((1, _D), lambda i: (0, 0)),
            pl.BlockSpec((1, _D), lambda i: (0, 0)),
        ],
        out_specs=pl.BlockSpec((_BLK, _D), lambda i: (i, 0)),
        out_shape=jax.ShapeDtypeStruct((_N, _D), jnp.float32),
    )(accp, degp, x, W, b, gamma, beta)


def kernel(x, edge_index, edge_weight, W, b, gamma, beta):
    src = edge_index[0].astype(jnp.int32)
    dst = edge_index[1].astype(jnp.int32)
    ew = edge_weight.astype(jnp.float32)

    pad = _EPAD - _E
    src3 = jnp.concatenate([src, jnp.zeros((pad,), jnp.int32)]).reshape(
        _NW, _NCHUNK, _CH)
    dst3 = jnp.concatenate([dst, jnp.zeros((pad,), jnp.int32)]).reshape(
        _NW, _NCHUNK, _CH)
    ew3 = jnp.concatenate([ew, jnp.full((pad,), -1.0, jnp.float32)]).reshape(
        _NW, _NCHUNK, _CH)

    degp = _deg_sc(dst3, ew3)
    y = _y_tc(degp, x)
    accp = _agg_sc(src3, dst3, ew3, y)
    return _out_tc(accp, degp, x, W, b.reshape(1, _D),
                   gamma.reshape(1, _D), beta.reshape(1, _D))


# speedup vs baseline: 10.0523x; 10.0523x over previous
import dataclasses
import functools

import jax
import jax.numpy as jnp
from jax import lax
from jax.experimental import pallas as pl
from jax.experimental.pallas import tpu as pltpu
from jax.experimental.pallas import tpu_sc as plsc

_N = 10000
_E = 320000
_D = 128

_NC = 2
_NS = 16
_NW = _NC * _NS

_CH = 128
_NCHUNK = 80
_EPT = _CH * _NCHUNK
_EPAD = _EPT * _NW
_NPAD = 10240
_RPT = _NPAD // _NS
_ZROWS = 128

_mesh = plsc.VectorSubcoreMesh(core_axis_name="c", subcore_axis_name="s")

_sc_params = pltpu.CompilerParams()
if "needs_layout_passes" in pltpu.CompilerParams.__dataclass_fields__:
    _sc_params = dataclasses.replace(_sc_params, needs_layout_passes=False)


@functools.partial(
    pl.kernel,
    out_type=jax.ShapeDtypeStruct((_NC, _NPAD, 16), jnp.float32),
    mesh=_mesh,
    scratch_types=[
        pltpu.VMEM((_NCHUNK, _CH), jnp.int32),
        pltpu.VMEM((_NCHUNK, _CH), jnp.float32),
        pltpu.VMEM((_CH, 16), jnp.float32),
        pltpu.VMEM_SHARED((_NPAD, 16), jnp.float32),
        pltpu.SemaphoreType.DMA,
    ],
    compiler_params=_sc_params,
)
def _deg_sc(dst_hbm, ew_hbm, out_hbm, dstv, ewv, rows, acc, sem):
    cid = lax.axis_index("c")
    sid = lax.axis_index("s")
    wid = cid * _NS + sid

    @pl.loop(0, _ZROWS)
    def _(i):
        rows[i, :] = jnp.zeros((16,), jnp.float32)

    @pl.loop(0, _RPT // _ZROWS)
    def _(j):
        pltpu.sync_copy(
            rows.at[pl.ds(0, _ZROWS)],
            acc.at[pl.ds(sid * _RPT + j * _ZROWS, _ZROWS)],
        )

    plsc.subcore_barrier()

    pltpu.sync_copy(dst_hbm.at[wid], dstv)
    pltpu.sync_copy(ew_hbm.at[wid], ewv)

    @pl.loop(0, _NCHUNK)
    def _(k):
        @pl.loop(0, _CH)
        def _(e):
            w = plsc.load_gather(
                ewv,
                [jnp.full((16,), k, jnp.int32), jnp.full((16,), e, jnp.int32)],
            )
            rows[e, :] = (w + 1.0) * 0.5

        pltpu.sync_copy(rows, acc.at[dstv.at[k]], add=True)

    plsc.subcore_barrier()
    pltpu.sync_copy(
        acc.at[pl.ds(sid * _RPT, _RPT)],
        out_hbm.at[cid, pl.ds(sid * _RPT, _RPT)],
    )


@functools.partial(
    pl.kernel,
    out_type=jax.ShapeDtypeStruct((_NC, _NPAD, _D), jnp.float32),
    mesh=_mesh,
    scratch_types=[
        pltpu.VMEM((_NCHUNK, _CH), jnp.int32),
        pltpu.VMEM((_NCHUNK, _CH), jnp.int32),
        pltpu.VMEM((_NCHUNK, _CH), jnp.float32),
        pltpu.VMEM((_CH, _D), jnp.float32),
        pltpu.VMEM_SHARED((_NPAD, _D), jnp.float32),
        pltpu.SemaphoreType.DMA,
    ],
    compiler_params=_sc_params,
)
def _agg_sc(src_hbm, dst_hbm, ew_hbm, y_hbm, out_hbm,
            srcv, dstv, ewv, rows, acc, sem):
    cid = lax.axis_index("c")
    sid = lax.axis_index("s")
    wid = cid * _NS + sid

    @pl.loop(0, _ZROWS)
    def _(i):
        for j in range(_D // 16):
            rows[i, pl.ds(j * 16, 16)] = jnp.zeros((16,), jnp.float32)

    @pl.loop(0, _RPT // _ZROWS)
    def _(j):
        pltpu.sync_copy(
            rows.at[pl.ds(0, _ZROWS)],
            acc.at[pl.ds(sid * _RPT + j * _ZROWS, _ZROWS)],
        )

    plsc.subcore_barrier()

    pltpu.sync_copy(src_hbm.at[wid], srcv)
    pltpu.sync_copy(dst_hbm.at[wid], dstv)
    pltpu.sync_copy(ew_hbm.at[wid], ewv)

    @pl.loop(0, _NCHUNK)
    def _(k):
        pltpu.async_copy(y_hbm.at[srcv.at[k]], rows, sem).wait()

        @pl.loop(0, _CH)
        def _(e):
            w = plsc.load_gather(
                ewv,
                [jnp.full((16,), k, jnp.int32), jnp.full((16,), e, jnp.int32)],
            )
            w = (w + 1.0) * 0.5
            for j in range(_D // 16):
                sl = pl.ds(j * 16, 16)
                rows[e, sl] = rows[e, sl] * w

        pltpu.sync_copy(rows, acc.at[dstv.at[k]], add=True)

    plsc.subcore_barrier()
    pltpu.sync_copy(
        acc.at[pl.ds(sid * _RPT, _RPT)],
        out_hbm.at[cid, pl.ds(sid * _RPT, _RPT)],
    )


_BLK = 1000


def _y_body(degp_ref, x_ref, y_ref):
    deg = 1.0 + degp_ref[0, :, 0:1] + degp_ref[1, :, 0:1]
    y_ref[...] = x_ref[...] * lax.rsqrt(deg)


def _y_tc(degp, x):
    return pl.pallas_call(
        _y_body,
        grid=(_N // _BLK,),
        in_specs=[
            pl.BlockSpec((_NC, _BLK, 16), lambda i: (0, i, 0)),
            pl.BlockSpec((_BLK, _D), lambda i: (i, 0)),
        ],
        out_specs=pl.BlockSpec((_BLK, _D), lambda i: (i, 0)),
        out_shape=jax.ShapeDtypeStruct((_N, _D), jnp.float32),
    )(degp, x)


def _out_body(accp_ref, degp_ref, x_ref, w_ref, b_ref, g_ref, bt_ref, o_ref):
    deg = 1.0 + degp_ref[0, :, 0:1] + degp_ref[1, :, 0:1]
    dinv = lax.rsqrt(deg)
    z = (accp_ref[0] + accp_ref[1]) * dinv + x_ref[...] * (1.0 / deg)
    h = jnp.dot(z, w_ref[...], preferred_element_type=jnp.float32) + b_ref[...]
    h = jnp.where(h >= 0.0, h, 0.01 * h)
    mu = jnp.mean(h, axis=-1, keepdims=True)
    var = jnp.mean((h - mu) ** 2, axis=-1, keepdims=True)
    o_ref[...] = (h - mu) * lax.rsqrt(var + 1e-5) * g_ref[...] + bt_ref[...]


def _out_tc(accp, degp, x, W, b, gamma, beta):
    return pl.pallas_call(
        _out_body,
        grid=(_N // _BLK,),
        in_specs=[
            pl.BlockSpec((_NC, _BLK, _D), lambda i: (0, i, 0)),
            pl.BlockSpec((_NC, _BLK, 16), lambda i: (0, i, 0)),
            pl.BlockSpec((_BLK, _D), lambda i: (i, 0)),
            pl.BlockSpec((_D, _D), lambda i: (0, 0)),
            pl.BlockSpec((1, _D), lambda i: (0, 0)),
            pl.BlockSpec((1, _D), lambda i: (0, 0)),
            pl.BlockSpec((1, _D), lambda i: (0, 0)),
        ],
        out_specs=pl.BlockSpec((_BLK, _D), lambda i: (i, 0)),
        out_shape=jax.ShapeDtypeStruct((_N, _D), jnp.float32),
    )(accp, degp, x, W, b, gamma, beta)


def kernel(x, edge_index, edge_weight, W, b, gamma, beta):
    src = edge_index[0].astype(jnp.int32)
    dst = edge_index[1].astype(jnp.int32)
    ew = edge_weight.astype(jnp.float32)

    pad = _EPAD - _E
    src3 = jnp.concatenate([src, jnp.zeros((pad,), jnp.int32)]).reshape(
        _NW, _NCHUNK, _CH)
    dst3 = jnp.concatenate([dst, jnp.zeros((pad,), jnp.int32)]).reshape(
        _NW, _NCHUNK, _CH)
    ew3 = jnp.concatenate([ew, jnp.full((pad,), -1.0, jnp.float32)]).reshape(
        _NW, _NCHUNK, _CH)

    degp = _deg_sc(dst3, ew3)
    y = _y_tc(degp, x)
    accp = _agg_sc(src3, dst3, ew3, y)
    return _out_tc(accp, degp, x, W, b.reshape(1, _D),
                   gamma.reshape(1, _D), beta.reshape(1, _D))


# speedup vs baseline: 10.4648x; 1.0410x over previous
import dataclasses
import functools

import jax
import jax.numpy as jnp
from jax import lax
from jax.experimental import pallas as pl
from jax.experimental.pallas import tpu as pltpu
from jax.experimental.pallas import tpu_sc as plsc

_N = 10000
_E = 320000
_D = 128

_NC = 2
_NS = 16
_NW = _NC * _NS

_CH = 128
_NCHUNK = 80
_EPT = _CH * _NCHUNK
_EPAD = _EPT * _NW
_NPAD = 10240
_RPT = _NPAD // _NS
_ZROWS = 128

_mesh = plsc.VectorSubcoreMesh(core_axis_name="c", subcore_axis_name="s")

_sc_params = pltpu.CompilerParams()
if "needs_layout_passes" in pltpu.CompilerParams.__dataclass_fields__:
    _sc_params = dataclasses.replace(_sc_params, needs_layout_passes=False)


@functools.partial(
    pl.kernel,
    out_type=jax.ShapeDtypeStruct((_NC, _NPAD, 16), jnp.float32),
    mesh=_mesh,
    scratch_types=[
        pltpu.VMEM((_NCHUNK, _CH), jnp.int32),
        pltpu.VMEM((_NCHUNK, _CH), jnp.float32),
        pltpu.VMEM((_CH, 16), jnp.float32),
        pltpu.VMEM_SHARED((_NPAD, 16), jnp.float32),
        pltpu.SemaphoreType.DMA,
    ],
    compiler_params=_sc_params,
)
def _deg_sc(dst_hbm, ew_hbm, out_hbm, dstv, ewv, rows, acc, sem):
    cid = lax.axis_index("c")
    sid = lax.axis_index("s")
    wid = cid * _NS + sid

    @pl.loop(0, _ZROWS)
    def _(i):
        rows[i, :] = jnp.zeros((16,), jnp.float32)

    @pl.loop(0, _RPT // _ZROWS)
    def _(j):
        pltpu.sync_copy(
            rows.at[pl.ds(0, _ZROWS)],
            acc.at[pl.ds(sid * _RPT + j * _ZROWS, _ZROWS)],
        )

    pltpu.sync_copy(dst_hbm.at[wid], dstv)
    pltpu.sync_copy(ew_hbm.at[wid], ewv)

    plsc.subcore_barrier()

    @pl.loop(0, _NCHUNK)
    def _(k):
        @pl.loop(0, _CH)
        def _(e):
            w = plsc.load_gather(
                ewv,
                [jnp.full((16,), k, jnp.int32), jnp.full((16,), e, jnp.int32)],
            )
            rows[e, :] = (w + 1.0) * 0.5

        pltpu.sync_copy(rows, acc.at[dstv.at[k]], add=True)

    plsc.subcore_barrier()
    pltpu.sync_copy(
        acc.at[pl.ds(sid * _RPT, _RPT)],
        out_hbm.at[cid, pl.ds(sid * _RPT, _RPT)],
    )


@functools.partial(
    pl.kernel,
    out_type=jax.ShapeDtypeStruct((_NC, _NPAD, _D), jnp.float32),
    mesh=_mesh,
    scratch_types=[
        pltpu.VMEM((16, _CH), jnp.int32),
        pltpu.VMEM((16, _CH), jnp.int32),
        pltpu.VMEM((16, _CH), jnp.float32),
        pltpu.VMEM((_CH, _D), jnp.float32),
        pltpu.VMEM((_CH, _D), jnp.float32),
        pltpu.VMEM_SHARED((_NPAD, _D), jnp.float32),
        pltpu.SemaphoreType.DMA,
        pltpu.SemaphoreType.DMA,
        pltpu.SemaphoreType.DMA,
        pltpu.SemaphoreType.DMA,
        pltpu.SemaphoreType.DMA,
    ],
    compiler_params=_sc_params,
)
def _agg_sc(src_hbm, dst_hbm, ew_hbm, y_hbm, out_hbm,
            sBuf, dBuf, eBuf, r0, r1, acc, si, sg0, sg1, ss0, ss1):
    cid = lax.axis_index("c")
    sid = lax.axis_index("s")
    wid = cid * _NS + sid
    rowsb = (r0, r1)
    sg = (sg0, sg1)
    ss = (ss0, ss1)

    @pl.loop(0, _ZROWS)
    def _(i):
        for j in range(_D // 16):
            r0[i, pl.ds(j * 16, 16)] = jnp.zeros((16,), jnp.float32)

    @pl.loop(0, _RPT // _ZROWS)
    def _(j):
        pltpu.sync_copy(
            r0.at[pl.ds(0, _ZROWS)],
            acc.at[pl.ds(sid * _RPT + j * _ZROWS, _ZROWS)],
        )

    pltpu.sync_copy(src_hbm.at[wid, pl.ds(0, 8)], sBuf.at[pl.ds(0, 8)])
    pltpu.sync_copy(dst_hbm.at[wid, pl.ds(0, 8)], dBuf.at[pl.ds(0, 8)])
    pltpu.sync_copy(ew_hbm.at[wid, pl.ds(0, 8)], eBuf.at[pl.ds(0, 8)])

    plsc.subcore_barrier()

    pltpu.async_copy(y_hbm.at[sBuf.at[0]], r0, sg0)

    def refill_issue(w_next, half):
        nb = (1 - half) * 8
        hb = pl.multiple_of(w_next * 8, 8)
        pltpu.async_copy(src_hbm.at[wid, pl.ds(hb, 8)],
                         sBuf.at[pl.ds(nb, 8)], si)
        pltpu.async_copy(dst_hbm.at[wid, pl.ds(hb, 8)],
                         dBuf.at[pl.ds(nb, 8)], si)
        pltpu.async_copy(ew_hbm.at[wid, pl.ds(hb, 8)],
                         eBuf.at[pl.ds(nb, 8)], si)

    def refill_drain(half):
        nb = (1 - half) * 8
        pltpu.make_async_copy(src_hbm.at[wid, pl.ds(0, 8)],
                              sBuf.at[pl.ds(nb, 8)], si).wait()
        pltpu.make_async_copy(dst_hbm.at[wid, pl.ds(0, 8)],
                              dBuf.at[pl.ds(nb, 8)], si).wait()
        pltpu.make_async_copy(ew_hbm.at[wid, pl.ds(0, 8)],
                              eBuf.at[pl.ds(nb, 8)], si).wait()

    def chunk(half, c, w_next=None, first=False, last=False):
        b = c % 2
        buf = rowsb[b]
        row = half * 8 + c

        pltpu.make_async_copy(y_hbm.at[sBuf.at[row]], buf, sg[b]).wait()

        @pl.loop(0, _CH)
        def _(e):
            w = plsc.load_gather(
                eBuf,
                [jnp.full((16,), row, jnp.int32),
                 jnp.full((16,), e, jnp.int32)],
            )
            w = (w + 1.0) * 0.5
            for j in range(_D // 16):
                sl = pl.ds(j * 16, 16)
                buf[e, sl] = buf[e, sl] * w

        if not first:
            pltpu.make_async_copy(
                rowsb[1 - b], acc.at[dBuf.at[row]], ss[1 - b]).wait()

        pltpu.async_copy(buf, acc.at[dBuf.at[row]], ss[b], add=True)

        if not last:
            nrow = row + 1 if c < 7 else (1 - half) * 8
            pltpu.async_copy(y_hbm.at[sBuf.at[nrow]], rowsb[1 - b],
                             sg[1 - b])
        if w_next is not None and c == 2:
            refill_issue(w_next, half)
        if w_next is not None and c == 6:
            refill_drain(half)

    def window(w, half, w_next, first=False, last=False):
        for c in range(8):
            chunk(half, c, w_next=w_next,
                  first=first and c == 0, last=last and c == 7)

    window(0, 0, 1, first=True)
    window(1, 1, 2)

    @pl.loop(2, 8, step=2)
    def _(w0):
        window(w0, 0, w0 + 1)
        window(w0 + 1, 1, w0 + 2)

    window(8, 0, 9)
    window(9, 1, None, last=True)

    pltpu.make_async_copy(
        rowsb[(_NCHUNK - 1) % 2], acc.at[dBuf.at[0]],
        ss[(_NCHUNK - 1) % 2]).wait()

    plsc.subcore_barrier()
    pltpu.sync_copy(
        acc.at[pl.ds(sid * _RPT, _RPT)],
        out_hbm.at[cid, pl.ds(sid * _RPT, _RPT)],
    )


_BLK = 1000


def _y_body(degp_ref, x_ref, y_ref):
    deg = 1.0 + degp_ref[0, :, 0:1] + degp_ref[1, :, 0:1]
    y_ref[...] = x_ref[...] * lax.rsqrt(deg)


def _y_tc(degp, x):
    return pl.pallas_call(
        _y_body,
        grid=(_N // _BLK,),
        in_specs=[
            pl.BlockSpec((_NC, _BLK, 16), lambda i: (0, i, 0)),
            pl.BlockSpec((_BLK, _D), lambda i: (i, 0)),
        ],
        out_specs=pl.BlockSpec((_BLK, _D), lambda i: (i, 0)),
        out_shape=jax.ShapeDtypeStruct((_N, _D), jnp.float32),
    )(degp, x)


def _out_body(accp_ref, degp_ref, x_ref, w_ref, b_ref, g_ref, bt_ref, o_ref):
    deg = 1.0 + degp_ref[0, :, 0:1] + degp_ref[1, :, 0:1]
    dinv = lax.rsqrt(deg)
    z = (accp_ref[0] + accp_ref[1]) * dinv + x_ref[...] * (1.0 / deg)
    h = jnp.dot(z, w_ref[...], preferred_element_type=jnp.float32) + b_ref[...]
    h = jnp.where(h >= 0.0, h, 0.01 * h)
    mu = jnp.mean(h, axis=-1, keepdims=True)
    var = jnp.mean((h - mu) ** 2, axis=-1, keepdims=True)
    o_ref[...] = (h - mu) * lax.rsqrt(var + 1e-5) * g_ref[...] + bt_ref[...]


def _out_tc(accp, degp, x, W, b, gamma, beta):
    return pl.pallas_call(
        _out_body,
        grid=(_N // _BLK,),
        in_specs=[
            pl.BlockSpec((_NC, _BLK, _D), lambda i: (0, i, 0)),
            pl.BlockSpec((_NC, _BLK, 16), lambda i: (0, i, 0)),
            pl.BlockSpec((_BLK, _D), lambda i: (i, 0)),
            pl.BlockSpec((_D, _D), lambda i: (0, 0)),
            pl.BlockSpec((1, _D), lambda i: (0, 0)),
            pl.BlockSpec((1, _D), lambda i: (0, 0)),
            pl.BlockSpec((1, _D), lambda i: (0, 0)),
        ],
        out_specs=pl.BlockSpec((_BLK, _D), lambda i: (i, 0)),
        out_shape=jax.ShapeDtypeStruct((_N, _D), jnp.float32),
    )(accp, degp, x, W, b, gamma, beta)


def kernel(x, edge_index, edge_weight, W, b, gamma, beta):
    src = edge_index[0].astype(jnp.int32)
    dst = edge_index[1].astype(jnp.int32)
    ew = edge_weight.astype(jnp.float32)

    pad = _EPAD - _E
    src3 = jnp.concatenate([src, jnp.zeros((pad,), jnp.int32)]).reshape(
        _NW, _NCHUNK, _CH)
    dst3 = jnp.concatenate([dst, jnp.zeros((pad,), jnp.int32)]).reshape(
        _NW, _NCHUNK, _CH)
    ew3 = jnp.concatenate([ew, jnp.full((pad,), -1.0, jnp.float32)]).reshape(
        _NW, _NCHUNK, _CH)

    degp = _deg_sc(dst3, ew3)
    y = _y_tc(degp, x)
    accp = _agg_sc(src3, dst3, ew3, y)
    return _out_tc(accp, degp, x, W, b.reshape(1, _D),
                   gamma.reshape(1, _D), beta.reshape(1, _D))
